# Initial kernel scaffold; baseline (speedup 1.0000x reference)
import jax
import jax.numpy as jnp
from jax.experimental import pallas as pl


def kernel(pred_logits, rsrp_pred, gamma_true):
    raise NotImplementedError("write your pallas kernel here")


# speedup vs baseline: 1.9748x; 1.9748x over previous
import functools

import jax
import jax.numpy as jnp
from jax import lax
from jax.experimental import pallas as pl
from jax.experimental.pallas import tpu as pltpu

_LAMBDA = 0.5
_K = 32
_TAU = 0.8
_B = 128
_N = 8192
_BLK = 8
_GRID = _B // _BLK
_IMIN = -2147483648


def _loss_body(p_ref, r_ref, g_ref, out_ref, acc_ref):
    i = pl.program_id(0)

    @pl.when(i == 0)
    def _init():
        acc_ref[0] = 0.0
        acc_ref[1] = 0.0
        acc_ref[2] = 0.0
        acc_ref[3] = 0.0

    g = g_ref[...]
    p = p_ref[...]
    inv_tau = jnp.float32(1.0 / _TAU)

    gmax = jnp.max(g, axis=1, keepdims=True)
    pmax = jnp.max(p, axis=1, keepdims=True)
    eg = jnp.exp((g - gmax) * inv_tau)
    ep = jnp.exp((p - pmax) * inv_tau)
    zg = jnp.sum(eg, axis=1, keepdims=True)
    zp = jnp.sum(ep, axis=1, keepdims=True)
    s_raw = jnp.sum(eg * (g - p), axis=1, keepdims=True)

    b = lax.bitcast_convert_type(g, jnp.int32)
    key = b ^ lax.shift_right_logical(jnp.right_shift(b, 31), 1)

    nneg = jnp.sum((key >= 0).astype(jnp.int32), axis=1, keepdims=True)
    prefix0 = jnp.where(nneg >= _K, jnp.int32(0), jnp.int32(_IMIN))

    def _bit_step(t, prefix):
        bit = jnp.int32(30) - t
        cand = prefix | lax.shift_left(jnp.int32(1), bit)
        cnt = jnp.sum((key >= cand).astype(jnp.int32), axis=1, keepdims=True)
        return jnp.where(cnt >= _K, cand, prefix)

    thr = lax.fori_loop(0, 31, _bit_step, prefix0)

    mask = key >= thr
    cnt_row = jnp.sum(mask.astype(jnp.float32), axis=1, keepdims=True)
    d = p - g
    mse = jnp.sum(jnp.where(mask, d * d, jnp.float32(0.0)))

    link = jnp.sum((r_ref[...] - gmax) ** 2)
    kl = jnp.sum(s_raw / (zg * _TAU) + (pmax - gmax) * inv_tau
                 + jnp.log(zp / zg))

    acc_ref[0] += mse
    acc_ref[1] += jnp.sum(cnt_row)
    acc_ref[2] += link
    acc_ref[3] += kl

    @pl.when(i == _GRID - 1)
    def _fin():
        total = (acc_ref[0] / jnp.maximum(acc_ref[1], 1.0)
                 + _LAMBDA * acc_ref[2] / _B
                 + (_TAU * _TAU / _B) * acc_ref[3])
        out_ref[...] = total.reshape((1, 1))


@jax.jit
def kernel(pred_logits, rsrp_pred, gamma_true):
    out = pl.pallas_call(
        _loss_body,
        grid=(_GRID,),
        in_specs=[
            pl.BlockSpec((_BLK, _N), lambda i: (i, 0)),
            pl.BlockSpec((_BLK, 1), lambda i: (i, 0)),
            pl.BlockSpec((_BLK, _N), lambda i: (i, 0)),
        ],
        out_specs=pl.BlockSpec((1, 1), lambda i: (0, 0)),
        out_shape=jax.ShapeDtypeStruct((1, 1), jnp.float32),
        scratch_shapes=[pltpu.SMEM((4,), jnp.float32)],
    )(pred_logits, rsrp_pred, gamma_true)
    return out[0, 0]
